# Initial kernel scaffold; baseline (speedup 1.0000x reference)
import jax
import jax.numpy as jnp
from jax.experimental import pallas as pl


def kernel(boxes, scores):
    raise NotImplementedError("write your pallas kernel here")


# speedup vs baseline: 17.5356x; 17.5356x over previous
import functools

import jax
import jax.numpy as jnp
from jax import lax
from jax.experimental import pallas as pl
from jax.experimental.pallas import tpu as pltpu

_B = 128
_T = 0.5
_EPS = 1e-9


def _nms_body(btr_ref, btc_ref, ss_ref, ks_ref, kp_ref,
              x1s, y1s, x2s, y2s, ars, kv, ms):
    np_ = btr_ref.shape[1]
    nb = np_ // _B

    b0 = btr_ref[0, :]
    b1 = btr_ref[1, :]
    b2 = btr_ref[2, :]
    b3 = btr_ref[3, :]
    x1 = jnp.minimum(b0, b2)
    y1 = jnp.minimum(b1, b3)
    x2 = jnp.maximum(b0, b2)
    y2 = jnp.maximum(b1, b3)
    x1s[:] = x1
    y1s[:] = y1
    x2s[:] = x2
    y2s[:] = y2
    ars[:] = (x2 - x1) * (y2 - y1)
    kv[:] = jnp.ones((np_,), jnp.float32)

    lane = lax.broadcasted_iota(jnp.int32, (_B,), 0)
    iot_r = lax.broadcasted_iota(jnp.int32, (_B, _B), 0)
    iot_c = lax.broadcasted_iota(jnp.int32, (_B, _B), 1)

    def block_step(b, _):
        s = b * _B
        bc = btc_ref[pl.ds(s, _B), :]
        bx1 = jnp.minimum(bc[:, 0:1], bc[:, 2:3])
        by1 = jnp.minimum(bc[:, 1:2], bc[:, 3:4])
        bx2 = jnp.maximum(bc[:, 0:1], bc[:, 2:3])
        by2 = jnp.maximum(bc[:, 1:2], bc[:, 3:4])
        bar = (bx2 - bx1) * (by2 - by1)

        def iou_gt(rx1, ry1, rx2, ry2, rar):
            xx1 = jnp.maximum(bx1, rx1[None, :])
            yy1 = jnp.maximum(by1, ry1[None, :])
            xx2 = jnp.minimum(bx2, rx2[None, :])
            yy2 = jnp.minimum(by2, ry2[None, :])
            w = jnp.clip(xx2 - xx1, 0.0)
            h = jnp.clip(yy2 - yy1, 0.0)
            inter = w * h
            union = bar + rar[None, :] - inter
            return (inter / (union + _EPS)) > _T

        m = iou_gt(x1s[pl.ds(s, _B)], y1s[pl.ds(s, _B)],
                   x2s[pl.ds(s, _B)], y2s[pl.ds(s, _B)], ars[pl.ds(s, _B)])
        m01 = jnp.where(m & (iot_c > iot_r), 1.0, 0.0)
        ms[:, :] = m01

        def inner(k, kb):
            kbk = jnp.sum(kb * jnp.where(lane == k, 1.0, 0.0))
            row = jnp.reshape(ms[pl.ds(k, 1), :], (_B,))
            return kb * (1.0 - kbk * row)

        kb = lax.fori_loop(0, _B, inner, kv[pl.ds(s, _B)], unroll=4)
        kv[pl.ds(s, _B)] = kb
        kbc = jnp.reshape(kb, (_B, 1))

        def cross(c, _):
            cs = c * _B
            mc = iou_gt(x1s[pl.ds(cs, _B)], y1s[pl.ds(cs, _B)],
                        x2s[pl.ds(cs, _B)], y2s[pl.ds(cs, _B)],
                        ars[pl.ds(cs, _B)])
            sup = jnp.max(jnp.where(mc, kbc, 0.0), axis=0)
            kv[pl.ds(cs, _B)] = kv[pl.ds(cs, _B)] * (1.0 - sup)
            return 0

        lax.fori_loop(b + 1, nb, cross, 0)
        return 0

    lax.fori_loop(0, nb, block_step, 0)
    kp_ref[:] = kv[:]
    ks_ref[:] = ss_ref[:] * kv[:]


def kernel(boxes, scores):
    n = scores.shape[0]
    order = jnp.argsort(-scores)
    sb = boxes[order]
    ss = scores[order]
    np_ = ((n + _B - 1) // _B) * _B
    pad = np_ - n
    sb_p = jnp.pad(sb, ((0, pad), (0, 0)))
    ss_p = jnp.pad(ss, ((0, pad),))

    ks_p, kp_p = pl.pallas_call(
        _nms_body,
        out_shape=[
            jax.ShapeDtypeStruct((np_,), jnp.float32),
            jax.ShapeDtypeStruct((np_,), jnp.float32),
        ],
        scratch_shapes=[
            pltpu.VMEM((np_,), jnp.float32),
            pltpu.VMEM((np_,), jnp.float32),
            pltpu.VMEM((np_,), jnp.float32),
            pltpu.VMEM((np_,), jnp.float32),
            pltpu.VMEM((np_,), jnp.float32),
            pltpu.VMEM((np_,), jnp.float32),
            pltpu.VMEM((_B, _B), jnp.float32),
        ],
    )(sb_p.T, sb_p, ss_p)

    keep = kp_p[:n] > 0.5
    return ks_p[:n], order, keep


# speedup vs baseline: 22.6333x; 1.2907x over previous
import functools

import jax
import jax.numpy as jnp
from jax import lax
from jax.experimental import pallas as pl
from jax.experimental.pallas import tpu as pltpu

_B = 128
_CH = 512
_T = 0.5
_EPS = 1e-9


def _nms_body(btr_ref, btc_ref, ss_ref, ks_ref, kp_ref,
              x1s, y1s, x2s, y2s, ars, kv, ms):
    np_ = btr_ref.shape[1]
    nb = np_ // _B

    b0 = btr_ref[0, :]
    b1 = btr_ref[1, :]
    b2 = btr_ref[2, :]
    b3 = btr_ref[3, :]
    x1 = jnp.minimum(b0, b2)
    y1 = jnp.minimum(b1, b3)
    x2 = jnp.maximum(b0, b2)
    y2 = jnp.maximum(b1, b3)
    x1s[:] = x1
    y1s[:] = y1
    x2s[:] = x2
    y2s[:] = y2
    ars[:] = (x2 - x1) * (y2 - y1)
    kv[:] = jnp.ones((np_,), jnp.float32)

    iot_r = lax.broadcasted_iota(jnp.int32, (_B, _B), 0)
    iot_c = lax.broadcasted_iota(jnp.int32, (_B, _B), 1)

    def block_step(b, _):
        s = b * _B
        bc = btc_ref[pl.ds(s, _B), :]
        bx1 = jnp.minimum(bc[:, 0:1], bc[:, 2:3])
        by1 = jnp.minimum(bc[:, 1:2], bc[:, 3:4])
        bx2 = jnp.maximum(bc[:, 0:1], bc[:, 2:3])
        by2 = jnp.maximum(bc[:, 1:2], bc[:, 3:4])
        bar = (bx2 - bx1) * (by2 - by1)

        def iou_gt(rx1, ry1, rx2, ry2, rar):
            xx1 = jnp.maximum(bx1, rx1[None, :])
            yy1 = jnp.maximum(by1, ry1[None, :])
            xx2 = jnp.minimum(bx2, rx2[None, :])
            yy2 = jnp.minimum(by2, ry2[None, :])
            w = jnp.clip(xx2 - xx1, 0.0)
            h = jnp.clip(yy2 - yy1, 0.0)
            inter = w * h
            union = bar + rar[None, :] - inter
            return (inter / (union + _EPS)) > _T

        m = iou_gt(x1s[pl.ds(s, _B)], y1s[pl.ds(s, _B)],
                   x2s[pl.ds(s, _B)], y2s[pl.ds(s, _B)], ars[pl.ds(s, _B)])
        m01 = jnp.where(m & (iot_c > iot_r), 1.0, 0.0)
        ms[:, :] = m01

        kb = kv[pl.ds(s, _B)]
        for g in range(_B // 8):
            k0 = g * 8
            rows = ms[k0:k0 + 8, :]
            kb8 = jnp.reshape(lax.slice(kb, (k0,), (k0 + 8,)), (1, 8))
            m8 = rows[:, k0:k0 + 8]
            for i in range(8):
                bi = kb8[:, i:i + 1]
                kb8 = kb8 * (1.0 - bi * m8[i:i + 1, :])
            sup = jnp.max(jnp.reshape(kb8, (8, 1)) * rows, axis=0)
            kb = kb * (1.0 - sup)
        kv[pl.ds(s, _B)] = kb
        kbc = jnp.reshape(kb, (_B, 1))

        end = s + _B - 1

        def cross(c, _):
            cs = c * _CH
            mc = iou_gt(x1s[pl.ds(cs, _CH)], y1s[pl.ds(cs, _CH)],
                        x2s[pl.ds(cs, _CH)], y2s[pl.ds(cs, _CH)],
                        ars[pl.ds(cs, _CH)])
            gate = (lax.broadcasted_iota(jnp.int32, (_CH,), 0) + cs) > end
            sup = jnp.max(jnp.where(mc, kbc, 0.0), axis=0)
            sup = jnp.where(gate, sup, 0.0)
            kv[pl.ds(cs, _CH)] = kv[pl.ds(cs, _CH)] * (1.0 - sup)
            return 0

        lax.fori_loop((s + _B) // _CH, np_ // _CH, cross, 0)
        return 0

    lax.fori_loop(0, nb, block_step, 0)
    kp_ref[:] = kv[:]
    ks_ref[:] = ss_ref[:] * kv[:]


def kernel(boxes, scores):
    n = scores.shape[0]
    order = jnp.argsort(-scores)
    sb = boxes[order]
    ss = scores[order]
    np_ = ((n + _CH - 1) // _CH) * _CH
    pad = np_ - n
    sb_p = jnp.pad(sb, ((0, pad), (0, 0)))
    ss_p = jnp.pad(ss, ((0, pad),))

    ks_p, kp_p = pl.pallas_call(
        _nms_body,
        out_shape=[
            jax.ShapeDtypeStruct((np_,), jnp.float32),
            jax.ShapeDtypeStruct((np_,), jnp.float32),
        ],
        scratch_shapes=[
            pltpu.VMEM((np_,), jnp.float32),
            pltpu.VMEM((np_,), jnp.float32),
            pltpu.VMEM((np_,), jnp.float32),
            pltpu.VMEM((np_,), jnp.float32),
            pltpu.VMEM((np_,), jnp.float32),
            pltpu.VMEM((np_,), jnp.float32),
            pltpu.VMEM((_B, _B), jnp.float32),
        ],
    )(sb_p.T, sb_p, ss_p)

    keep = kp_p[:n] > 0.5
    return ks_p[:n], order, keep
